# initial kernel scaffold (unmeasured)
import functools

import jax
import jax.numpy as jnp
from jax import lax
from jax.experimental import pallas as pl
from jax.experimental.pallas import tpu as pltpu

N_DEV = 8
N_EXP = 32


def kernel(x, router_W, route_idx, expert_W):
    n_tok, d_model = x.shape
    e_loc, _, d_hidden = expert_W.shape

    x_bf = x.astype(jnp.bfloat16)
    router_bf = router_W.astype(jnp.bfloat16)
    w_bf = expert_W.astype(jnp.bfloat16)

    def body(xbf_ref, router_ref, idx_ref, wbf_ref, out_ref,
             comm_ref, send_sems, recv_sems):
        my = lax.axis_index("i")
        left = lax.rem(my + N_DEV - 1, N_DEV)
        right = lax.rem(my + 1, N_DEV)

        barrier = pltpu.get_barrier_semaphore()
        for nbr in (left, right):
            pl.semaphore_signal(barrier, inc=1, device_id=(nbr,),
                                device_id_type=pl.DeviceIdType.MESH)
        pl.semaphore_wait(barrier, 2)

        scores = jnp.dot(xbf_ref[:, :], router_ref[:, :],
                         preferred_element_type=jnp.float32)
        e0 = idx_ref[:, 0:1]
        e1 = idx_ref[:, 1:2]
        col = lax.broadcasted_iota(jnp.int32, (n_tok, N_EXP), 1)
        s0 = jnp.sum(jnp.where(col == e0, scores, 0.0), axis=1, keepdims=True)
        s1 = jnp.sum(jnp.where(col == e1, scores, 0.0), axis=1, keepdims=True)
        m = jnp.maximum(s0, s1)
        p0 = jnp.exp(s0 - m)
        p1 = jnp.exp(s1 - m)
        w0 = p0 / (p0 + p1)
        w1 = p1 / (p0 + p1)

        comm_ref[0] = wbf_ref[...]

        acc = jnp.zeros((n_tok, d_hidden), jnp.float32)
        for hop in range(N_DEV):
            if hop < N_DEV - 1:
                rdma = pltpu.make_async_remote_copy(
                    src_ref=comm_ref.at[hop],
                    dst_ref=comm_ref.at[hop + 1],
                    send_sem=send_sems.at[hop],
                    recv_sem=recv_sems.at[hop],
                    device_id=(right,),
                    device_id_type=pl.DeviceIdType.MESH,
                )
                rdma.start()
            src_dev = lax.rem(my + N_DEV - hop, N_DEV)
            for j in range(e_loc):
                eg = src_dev * e_loc + j
                w_e = (jnp.where(e0 == eg, w0, 0.0)
                       + jnp.where(e1 == eg, w1, 0.0))
                y = jnp.dot(xbf_ref[:, :], comm_ref[hop, j],
                            preferred_element_type=jnp.float32)
                acc = acc + w_e * y
            if hop < N_DEV - 1:
                rdma.wait()
        out_ref[...] = acc

        @functools.partial(pl.run_scoped,
                           second_barrier=pltpu.SemaphoreType.REGULAR)
        def _(second_barrier):
            for nbr in (left, right):
                pl.semaphore_signal(second_barrier, inc=1, device_id=(nbr,),
                                    device_id_type=pl.DeviceIdType.MESH)
            pl.semaphore_wait(second_barrier, 2)

    return pl.pallas_call(
        body,
        out_shape=jax.ShapeDtypeStruct((n_tok, d_hidden), jnp.float32),
        in_specs=[pl.BlockSpec(memory_space=pltpu.VMEM)] * 4,
        out_specs=pl.BlockSpec(memory_space=pltpu.VMEM),
        scratch_shapes=[
            pltpu.VMEM((N_DEV, e_loc, d_model, d_hidden), jnp.bfloat16),
            pltpu.SemaphoreType.DMA((N_DEV - 1,)),
            pltpu.SemaphoreType.DMA((N_DEV - 1,)),
        ],
        compiler_params=pltpu.CompilerParams(collective_id=0),
    )(x_bf, router_bf, route_idx, w_bf)


# baseline (device time: 357101 ns/iter reference)
import functools

import jax
import jax.numpy as jnp
from jax import lax
from jax.experimental import pallas as pl
from jax.experimental.pallas import tpu as pltpu

N_DEV = 8
N_EXP = 32


def kernel(x, router_W, route_idx, expert_W):
    n_tok, d_model = x.shape
    e_loc, _, d_hidden = expert_W.shape

    x_bf = x.astype(jnp.bfloat16)
    router_bf = router_W.astype(jnp.bfloat16)
    w_bf = expert_W.astype(jnp.bfloat16)

    def body(xbf_ref, router_ref, idx_ref, wbf_ref, out_ref,
             comm_ref, send_sems, recv_sems):
        my = lax.axis_index("i")
        left = lax.rem(my + N_DEV - 1, N_DEV)
        right = lax.rem(my + 1, N_DEV)

        barrier = pltpu.get_barrier_semaphore()
        for nbr in (left, right):
            pl.semaphore_signal(barrier, inc=1, device_id=(nbr,),
                                device_id_type=pl.DeviceIdType.MESH)
        pl.semaphore_wait(barrier, 2)

        scores = jnp.dot(xbf_ref[:, :], router_ref[:, :],
                         preferred_element_type=jnp.float32)
        e0 = idx_ref[:, 0:1]
        e1 = idx_ref[:, 1:2]
        col = lax.broadcasted_iota(jnp.int32, (n_tok, N_EXP), 1)
        s0 = jnp.sum(jnp.where(col == e0, scores, 0.0), axis=1, keepdims=True)
        s1 = jnp.sum(jnp.where(col == e1, scores, 0.0), axis=1, keepdims=True)
        m = jnp.maximum(s0, s1)
        p0 = jnp.exp(s0 - m)
        p1 = jnp.exp(s1 - m)
        w0 = p0 / (p0 + p1)
        w1 = p1 / (p0 + p1)

        comm_ref[0] = wbf_ref[...]

        acc = jnp.zeros((n_tok, d_hidden), jnp.float32)
        for hop in range(N_DEV):
            if hop < N_DEV - 1:
                rdma = pltpu.make_async_remote_copy(
                    src_ref=comm_ref.at[hop],
                    dst_ref=comm_ref.at[hop + 1],
                    send_sem=send_sems.at[hop],
                    recv_sem=recv_sems.at[hop],
                    device_id=(right,),
                    device_id_type=pl.DeviceIdType.MESH,
                )
                rdma.start()
            src_dev = lax.rem(my + N_DEV - hop, N_DEV)
            for j in range(e_loc):
                eg = src_dev * e_loc + j
                w_e = (jnp.where(e0 == eg, w0, 0.0)
                       + jnp.where(e1 == eg, w1, 0.0))
                y = jnp.dot(xbf_ref[:, :], comm_ref[hop, j],
                            preferred_element_type=jnp.float32)
                acc = acc + w_e * y
            if hop < N_DEV - 1:
                rdma.wait()
        out_ref[...] = acc

        @functools.partial(pl.run_scoped,
                           second_barrier=pltpu.SemaphoreType.REGULAR)
        def _(second_barrier):
            for nbr in (left, right):
                pl.semaphore_signal(second_barrier, inc=1, device_id=(nbr,),
                                    device_id_type=pl.DeviceIdType.MESH)
            pl.semaphore_wait(second_barrier, 2)

    return pl.pallas_call(
        body,
        out_shape=jax.ShapeDtypeStruct((n_tok, d_hidden), jnp.float32),
        in_specs=[pl.BlockSpec(memory_space=pltpu.VMEM)] * 4,
        out_specs=pl.BlockSpec(memory_space=pltpu.VMEM),
        scratch_shapes=[
            pltpu.VMEM((N_DEV, e_loc, d_model, d_hidden), jnp.bfloat16),
            pltpu.SemaphoreType.DMA((N_DEV - 1,)),
            pltpu.SemaphoreType.DMA((N_DEV - 1,)),
        ],
        compiler_params=pltpu.CompilerParams(
            collective_id=0,
            vmem_limit_bytes=64 * 1024 * 1024,
        ),
    )(x_bf, router_bf, route_idx, w_bf)


# device time: 203993 ns/iter; 1.7506x vs baseline; 1.7506x over previous
import functools

import jax
import jax.numpy as jnp
from jax import lax
from jax.experimental import pallas as pl
from jax.experimental.pallas import tpu as pltpu

N_DEV = 8
N_EXP = 32


def kernel(x, router_W, route_idx, expert_W):
    n_tok, d_model = x.shape
    e_loc, _, d_hidden = expert_W.shape

    x_bf = x.astype(jnp.bfloat16)
    router_bf = router_W.astype(jnp.bfloat16)
    w_bf = expert_W.astype(jnp.bfloat16)

    def body(xbf_ref, router_ref, idx_ref, wbf_ref, out_ref,
             cw_ref, ccw_ref,
             cw_send_sems, cw_recv_sems, ccw_send_sems, ccw_recv_sems):
        my = lax.axis_index("i")
        left = lax.rem(my + N_DEV - 1, N_DEV)
        right = lax.rem(my + 1, N_DEV)

        barrier = pltpu.get_barrier_semaphore()
        for nbr in (left, right):
            pl.semaphore_signal(barrier, inc=1, device_id=(nbr,),
                                device_id_type=pl.DeviceIdType.MESH)
        pl.semaphore_wait(barrier, 2)

        scores = jnp.dot(xbf_ref[:, :], router_ref[:, :],
                         preferred_element_type=jnp.float32)
        e0 = idx_ref[:, 0:1]
        e1 = idx_ref[:, 1:2]
        col = lax.broadcasted_iota(jnp.int32, (n_tok, N_EXP), 1)
        s0 = jnp.sum(jnp.where(col == e0, scores, 0.0), axis=1, keepdims=True)
        s1 = jnp.sum(jnp.where(col == e1, scores, 0.0), axis=1, keepdims=True)
        m = jnp.maximum(s0, s1)
        p0 = jnp.exp(s0 - m)
        p1 = jnp.exp(s1 - m)
        w0 = p0 / (p0 + p1)
        w1 = p1 / (p0 + p1)

        half = e_loc // 2
        cw_ref[0] = wbf_ref[0:half]
        ccw_ref[0] = wbf_ref[half:e_loc]

        acc = jnp.zeros((n_tok, d_hidden), jnp.float32)
        for hop in range(N_DEV):
            rdmas = []
            if hop < N_DEV - 1:
                for buf, sems_s, sems_r, dst in (
                    (cw_ref, cw_send_sems, cw_recv_sems, right),
                    (ccw_ref, ccw_send_sems, ccw_recv_sems, left),
                ):
                    rdma = pltpu.make_async_remote_copy(
                        src_ref=buf.at[hop],
                        dst_ref=buf.at[hop + 1],
                        send_sem=sems_s.at[hop],
                        recv_sem=sems_r.at[hop],
                        device_id=(dst,),
                        device_id_type=pl.DeviceIdType.MESH,
                    )
                    rdma.start()
                    rdmas.append(rdma)
            src_cw = lax.rem(my + N_DEV - hop, N_DEV)
            src_ccw = lax.rem(my + hop, N_DEV)
            for j in range(half):
                for src_dev, buf, j_off in ((src_cw, cw_ref, 0),
                                            (src_ccw, ccw_ref, half)):
                    eg = src_dev * e_loc + j_off + j
                    w_e = (jnp.where(e0 == eg, w0, 0.0)
                           + jnp.where(e1 == eg, w1, 0.0))
                    y = jnp.dot(xbf_ref[:, :], buf[hop, j],
                                preferred_element_type=jnp.float32)
                    acc = acc + w_e * y
            for rdma in rdmas:
                rdma.wait()
        out_ref[...] = acc

        @functools.partial(pl.run_scoped,
                           second_barrier=pltpu.SemaphoreType.REGULAR)
        def _(second_barrier):
            for nbr in (left, right):
                pl.semaphore_signal(second_barrier, inc=1, device_id=(nbr,),
                                    device_id_type=pl.DeviceIdType.MESH)
            pl.semaphore_wait(second_barrier, 2)

    return pl.pallas_call(
        body,
        out_shape=jax.ShapeDtypeStruct((n_tok, d_hidden), jnp.float32),
        in_specs=[pl.BlockSpec(memory_space=pltpu.VMEM)] * 4,
        out_specs=pl.BlockSpec(memory_space=pltpu.VMEM),
        scratch_shapes=[
            pltpu.VMEM((N_DEV, e_loc // 2, d_model, d_hidden), jnp.bfloat16),
            pltpu.VMEM((N_DEV, e_loc // 2, d_model, d_hidden), jnp.bfloat16),
            pltpu.SemaphoreType.DMA((N_DEV - 1,)),
            pltpu.SemaphoreType.DMA((N_DEV - 1,)),
            pltpu.SemaphoreType.DMA((N_DEV - 1,)),
            pltpu.SemaphoreType.DMA((N_DEV - 1,)),
        ],
        compiler_params=pltpu.CompilerParams(
            collective_id=0,
            vmem_limit_bytes=64 * 1024 * 1024,
        ),
    )(x_bf, router_bf, route_idx, w_bf)


# device time: 161027 ns/iter; 2.2176x vs baseline; 1.2668x over previous
import functools

import jax
import jax.numpy as jnp
from jax import lax
from jax.experimental import pallas as pl
from jax.experimental.pallas import tpu as pltpu

N_DEV = 8
N_EXP = 32

MASKS = (1, 3, 4)
ORDERS = ((1, 3, 4), (3, 4, 1), (4, 1, 3))
COLS = ((0, 384), (384, 768), (768, 1024))


def kernel(x, router_W, route_idx, expert_W):
    n_tok, d_model = x.shape
    e_loc, _, d_hidden = expert_W.shape

    x_bf = x.astype(jnp.bfloat16)
    router_bf = router_W.astype(jnp.bfloat16)
    w_bf = expert_W.astype(jnp.bfloat16)

    def body(xbf_ref, router_ref, idx_ref, wbf_ref, out_ref,
             g0_ref, g1_ref, g2_ref, send_sems, recv_sems):
        my = lax.axis_index("i")
        gs = (g0_ref, g1_ref, g2_ref)

        barrier = pltpu.get_barrier_semaphore()
        for mask in MASKS:
            pl.semaphore_signal(barrier, inc=1, device_id=(my ^ mask,),
                                device_id_type=pl.DeviceIdType.MESH)
        pl.semaphore_wait(barrier, 3)

        scores = jnp.dot(xbf_ref[:, :], router_ref[:, :],
                         preferred_element_type=jnp.float32)
        e0 = idx_ref[:, 0:1]
        e1 = idx_ref[:, 1:2]
        col = lax.broadcasted_iota(jnp.int32, (n_tok, N_EXP), 1)
        s0 = jnp.sum(jnp.where(col == e0, scores, 0.0), axis=1, keepdims=True)
        s1 = jnp.sum(jnp.where(col == e1, scores, 0.0), axis=1, keepdims=True)
        m = jnp.maximum(s0, s1)
        p0 = jnp.exp(s0 - m)
        p1 = jnp.exp(s1 - m)
        w0 = p0 / (p0 + p1)
        w1 = p1 / (p0 + p1)

        for a in range(3):
            c0, c1 = COLS[a]
            gs[a][0] = wbf_ref[:, :, c0:c1]

        accs = [jnp.zeros((n_tok, COLS[a][1] - COLS[a][0]), jnp.float32)
                for a in range(3)]

        def compute(a, g):
            src_dev = my ^ g
            for j in range(e_loc):
                eg = src_dev * e_loc + j
                w_e = (jnp.where(e0 == eg, w0, 0.0)
                       + jnp.where(e1 == eg, w1, 0.0))
                y = jnp.dot(xbf_ref[:, :], gs[a][g, j],
                            preferred_element_type=jnp.float32)
                accs[a] = accs[a] + w_e * y

        descs = [[None] * 7 for _ in range(3)]

        def issue(a, step, pairs):
            mask = ORDERS[a][step]
            partner = my ^ mask
            for g, fi in pairs:
                rdma = pltpu.make_async_remote_copy(
                    src_ref=gs[a].at[g],
                    dst_ref=gs[a].at[g ^ mask],
                    send_sem=send_sems.at[a, fi],
                    recv_sem=recv_sems.at[a, fi],
                    device_id=(partner,),
                    device_id_type=pl.DeviceIdType.MESH,
                )
                rdma.start()
                descs[a][fi] = rdma

        for a in range(3):
            issue(a, 0, [(0, 0)])
        for a in range(3):
            compute(a, 0)
        for a in range(3):
            descs[a][0].wait_recv()

        for a in range(3):
            m0 = ORDERS[a][0]
            issue(a, 1, [(0, 1), (m0, 2)])
        for a in range(3):
            compute(a, ORDERS[a][0])
        for a in range(3):
            descs[a][1].wait_recv()
            descs[a][2].wait_recv()

        for a in range(3):
            m0, m1, _ = ORDERS[a]
            issue(a, 2, [(0, 3), (m0, 4)])
            issue(a, 2, [(m1, 5), (m0 ^ m1, 6)])
        for a in range(3):
            m0, m1, _ = ORDERS[a]
            compute(a, m1)
            compute(a, m0 ^ m1)
        for a in range(3):
            descs[a][3].wait_recv()
            descs[a][4].wait_recv()
        for a in range(3):
            m0, _, m2 = ORDERS[a]
            compute(a, m2)
            compute(a, m0 ^ m2)
        for a in range(3):
            descs[a][5].wait_recv()
            descs[a][6].wait_recv()
        for a in range(3):
            m0, m1, m2 = ORDERS[a]
            compute(a, m1 ^ m2)
            compute(a, m0 ^ m1 ^ m2)

        for a in range(3):
            c0, c1 = COLS[a]
            out_ref[:, c0:c1] = accs[a]

        for a in range(3):
            for fi in range(7):
                descs[a][fi].wait_send()

        @functools.partial(pl.run_scoped,
                           second_barrier=pltpu.SemaphoreType.REGULAR)
        def _(second_barrier):
            for mask in MASKS:
                pl.semaphore_signal(second_barrier, inc=1,
                                    device_id=(my ^ mask,),
                                    device_id_type=pl.DeviceIdType.MESH)
            pl.semaphore_wait(second_barrier, 3)

    return pl.pallas_call(
        body,
        out_shape=jax.ShapeDtypeStruct((n_tok, d_hidden), jnp.float32),
        in_specs=[pl.BlockSpec(memory_space=pltpu.VMEM)] * 4,
        out_specs=pl.BlockSpec(memory_space=pltpu.VMEM),
        scratch_shapes=[
            pltpu.VMEM((N_DEV, e_loc, d_model, 384), jnp.bfloat16),
            pltpu.VMEM((N_DEV, e_loc, d_model, 384), jnp.bfloat16),
            pltpu.VMEM((N_DEV, e_loc, d_model, 256), jnp.bfloat16),
            pltpu.SemaphoreType.DMA((3, 7)),
            pltpu.SemaphoreType.DMA((3, 7)),
        ],
        compiler_params=pltpu.CompilerParams(
            collective_id=0,
            vmem_limit_bytes=64 * 1024 * 1024,
        ),
    )(x_bf, router_bf, route_idx, w_bf)


# device time: 150093 ns/iter; 2.3792x vs baseline; 1.0728x over previous
import functools

import jax
import jax.numpy as jnp
from jax import lax
from jax.experimental import pallas as pl
from jax.experimental.pallas import tpu as pltpu

N_DEV = 8
N_EXP = 32

MASKS = (1, 3, 4)
ORDERS = ((1, 3, 4), (3, 4, 1), (4, 1, 3))
COLS = ((0, 384), (384, 768), (768, 1024))


def kernel(x, router_W, route_idx, expert_W):
    n_tok, d_model = x.shape
    e_loc, _, d_hidden = expert_W.shape

    x_bf = x.astype(jnp.bfloat16)
    router_bf = router_W.astype(jnp.bfloat16)
    w_bf = expert_W.astype(jnp.bfloat16)

    def body(xbf_ref, router_ref, idx_ref, wbf_ref, out_ref,
             g0_ref, g1_ref, g2_ref, send_sems, recv_sems):
        my = lax.axis_index("i")
        gs = (g0_ref, g1_ref, g2_ref)

        barrier = pltpu.get_barrier_semaphore()
        for mask in MASKS:
            pl.semaphore_signal(barrier, inc=1, device_id=(my ^ mask,),
                                device_id_type=pl.DeviceIdType.MESH)
        pl.semaphore_wait(barrier, 3)

        scores = jnp.dot(xbf_ref[:, :], router_ref[:, :],
                         preferred_element_type=jnp.float32)
        e0 = idx_ref[:, 0:1]
        e1 = idx_ref[:, 1:2]
        col = lax.broadcasted_iota(jnp.int32, (n_tok, N_EXP), 1)
        s0 = jnp.sum(jnp.where(col == e0, scores, 0.0), axis=1, keepdims=True)
        s1 = jnp.sum(jnp.where(col == e1, scores, 0.0), axis=1, keepdims=True)
        m = jnp.maximum(s0, s1)
        p0 = jnp.exp(s0 - m)
        p1 = jnp.exp(s1 - m)
        w0 = p0 / (p0 + p1)
        w1 = p1 / (p0 + p1)

        for a in range(3):
            c0, c1 = COLS[a]
            gs[a][0] = wbf_ref[:, :, c0:c1]

        accs = [jnp.zeros((n_tok, COLS[a][1] - COLS[a][0]), jnp.float32)
                for a in range(3)]

        def compute(a, g):
            src_dev = my ^ g
            for j in range(e_loc):
                eg = src_dev * e_loc + j
                w_e = (jnp.where(e0 == eg, w0, 0.0)
                       + jnp.where(e1 == eg, w1, 0.0))
                y = jnp.dot(xbf_ref[:, :], gs[a][g, j],
                            preferred_element_type=jnp.float32)
                accs[a] = accs[a] + w_e * y

        descs = [[None] * 7 for _ in range(3)]

        def issue(a, step, pairs):
            mask = ORDERS[a][step]
            partner = my ^ mask
            for g, fi in pairs:
                rdma = pltpu.make_async_remote_copy(
                    src_ref=gs[a].at[g],
                    dst_ref=gs[a].at[g ^ mask],
                    send_sem=send_sems.at[a, fi],
                    recv_sem=recv_sems.at[a, fi],
                    device_id=(partner,),
                    device_id_type=pl.DeviceIdType.MESH,
                )
                rdma.start()
                descs[a][fi] = rdma

        for a in range(3):
            issue(a, 0, [(0, 0)])
        for a in range(3):
            issue(a, 1, [(0, 1)])
            issue(a, 2, [(0, 3)])
        for a in range(3):
            compute(a, 0)
        for a in range(3):
            descs[a][0].wait_recv()

        for a in range(3):
            m0 = ORDERS[a][0]
            issue(a, 1, [(m0, 2)])
            issue(a, 2, [(m0, 4)])
        for a in range(3):
            compute(a, ORDERS[a][0])
        for a in range(3):
            descs[a][1].wait_recv()
            descs[a][2].wait_recv()

        for a in range(3):
            m0, m1, _ = ORDERS[a]
            issue(a, 2, [(m1, 5), (m0 ^ m1, 6)])
        for a in range(3):
            m0, m1, _ = ORDERS[a]
            compute(a, m1)
            compute(a, m0 ^ m1)
        for a in range(3):
            descs[a][3].wait_recv()
        for a in range(3):
            compute(a, ORDERS[a][2])
        for a in range(3):
            descs[a][4].wait_recv()
        for a in range(3):
            m0, _, m2 = ORDERS[a]
            compute(a, m0 ^ m2)
        for a in range(3):
            descs[a][5].wait_recv()
        for a in range(3):
            _, m1, m2 = ORDERS[a]
            compute(a, m1 ^ m2)
        for a in range(3):
            descs[a][6].wait_recv()
        for a in range(3):
            m0, m1, m2 = ORDERS[a]
            compute(a, m0 ^ m1 ^ m2)

        for a in range(3):
            c0, c1 = COLS[a]
            out_ref[:, c0:c1] = accs[a]

        for a in range(3):
            for fi in range(7):
                descs[a][fi].wait_send()

        @functools.partial(pl.run_scoped,
                           second_barrier=pltpu.SemaphoreType.REGULAR)
        def _(second_barrier):
            for mask in MASKS:
                pl.semaphore_signal(second_barrier, inc=1,
                                    device_id=(my ^ mask,),
                                    device_id_type=pl.DeviceIdType.MESH)
            pl.semaphore_wait(second_barrier, 3)

    return pl.pallas_call(
        body,
        out_shape=jax.ShapeDtypeStruct((n_tok, d_hidden), jnp.float32),
        in_specs=[pl.BlockSpec(memory_space=pltpu.VMEM)] * 4,
        out_specs=pl.BlockSpec(memory_space=pltpu.VMEM),
        scratch_shapes=[
            pltpu.VMEM((N_DEV, e_loc, d_model, 384), jnp.bfloat16),
            pltpu.VMEM((N_DEV, e_loc, d_model, 384), jnp.bfloat16),
            pltpu.VMEM((N_DEV, e_loc, d_model, 256), jnp.bfloat16),
            pltpu.SemaphoreType.DMA((3, 7)),
            pltpu.SemaphoreType.DMA((3, 7)),
        ],
        compiler_params=pltpu.CompilerParams(
            collective_id=0,
            vmem_limit_bytes=64 * 1024 * 1024,
        ),
    )(x_bf, router_bf, route_idx, w_bf)


# device time: 148954 ns/iter; 2.3974x vs baseline; 1.0076x over previous
import functools

import jax
import jax.numpy as jnp
from jax import lax
from jax.experimental import pallas as pl
from jax.experimental.pallas import tpu as pltpu

N_DEV = 8
N_EXP = 32

MASKS = (1, 3, 4)
ORDERS = ((1, 3, 4), (3, 4, 1), (4, 1, 3))
COLS = ((0, 384), (384, 768), (768, 1024))


def kernel(x, router_W, route_idx, expert_W):
    n_tok, d_model = x.shape
    e_loc, _, d_hidden = expert_W.shape

    x_bf16 = x.astype(jnp.bfloat16)
    router_bf16 = router_W.astype(jnp.bfloat16)
    w_bf16 = expert_W.astype(jnp.bfloat16)

    def body(x_ref, router_ref, idx_ref, w_ref, out_ref,
             g0_ref, g1_ref, g2_ref, send_sems, recv_sems):
        my = lax.axis_index("i")
        gs = (g0_ref, g1_ref, g2_ref)

        barrier = pltpu.get_barrier_semaphore()
        for mask in MASKS:
            pl.semaphore_signal(barrier, inc=1, device_id=(my ^ mask,),
                                device_id_type=pl.DeviceIdType.MESH)
        pl.semaphore_wait(barrier, 3)

        x_bf = x_ref[:, :]

        scores = jnp.dot(x_bf, router_ref[:, :],
                         preferred_element_type=jnp.float32)
        e0 = idx_ref[:, 0:1]
        e1 = idx_ref[:, 1:2]
        col = lax.broadcasted_iota(jnp.int32, (n_tok, N_EXP), 1)
        s0 = jnp.sum(jnp.where(col == e0, scores, 0.0), axis=1, keepdims=True)
        s1 = jnp.sum(jnp.where(col == e1, scores, 0.0), axis=1, keepdims=True)
        m = jnp.maximum(s0, s1)
        p0 = jnp.exp(s0 - m)
        p1 = jnp.exp(s1 - m)
        w0 = p0 / (p0 + p1)
        w1 = p1 / (p0 + p1)

        accs = [jnp.zeros((n_tok, COLS[a][1] - COLS[a][0]), jnp.float32)
                for a in range(3)]

        def compute(a, g, experts=range(e_loc)):
            src_dev = my ^ g
            for j in experts:
                eg = src_dev * e_loc + j
                w_e = (jnp.where(e0 == eg, w0, 0.0)
                       + jnp.where(e1 == eg, w1, 0.0))
                y = jnp.dot(x_bf, gs[a][g, j],
                            preferred_element_type=jnp.float32)
                accs[a] = accs[a] + w_e * y

        descs = [[None] * 8 for _ in range(3)]

        def issue(a, step, pairs):
            mask = ORDERS[a][step]
            partner = my ^ mask
            for g, fi in pairs:
                src = gs[a].at[g]
                dst = gs[a].at[g ^ mask]
                if fi >= 6:
                    j0, j1 = (0, 2) if fi == 6 else (2, e_loc)
                    src = gs[a].at[g, j0:j1]
                    dst = gs[a].at[g ^ mask, j0:j1]
                rdma = pltpu.make_async_remote_copy(
                    src_ref=src,
                    dst_ref=dst,
                    send_sem=send_sems.at[a, fi],
                    recv_sem=recv_sems.at[a, fi],
                    device_id=(partner,),
                    device_id_type=pl.DeviceIdType.MESH,
                )
                rdma.start()
                descs[a][fi] = rdma

        for a in range(3):
            c0, c1 = COLS[a]
            gs[a][0] = w_ref[:, :, c0:c1]
            issue(a, 0, [(0, 0)])
        for a in range(3):
            issue(a, 1, [(0, 1)])
            issue(a, 2, [(0, 3)])
        for a in range(3):
            compute(a, 0)
        for a in range(3):
            descs[a][0].wait_recv()

        for a in range(3):
            m0 = ORDERS[a][0]
            issue(a, 1, [(m0, 2)])
            issue(a, 2, [(m0, 4)])
        for a in range(3):
            compute(a, ORDERS[a][0])
        for a in range(3):
            descs[a][1].wait_recv()
            descs[a][2].wait_recv()

        for a in range(3):
            m0, m1, _ = ORDERS[a]
            issue(a, 2, [(m1, 5), (m0 ^ m1, 6), (m0 ^ m1, 7)])
        for a in range(3):
            m0, m1, _ = ORDERS[a]
            compute(a, m1)
            compute(a, m0 ^ m1)
        for a in range(3):
            descs[a][3].wait_recv()
        for a in range(3):
            compute(a, ORDERS[a][2])
        for a in range(3):
            descs[a][4].wait_recv()
        for a in range(3):
            m0, _, m2 = ORDERS[a]
            compute(a, m0 ^ m2)
        for a in range(3):
            descs[a][5].wait_recv()
        for a in range(3):
            _, m1, m2 = ORDERS[a]
            compute(a, m1 ^ m2)
        for a in range(3):
            descs[a][6].wait_recv()
        for a in range(3):
            m0, m1, m2 = ORDERS[a]
            compute(a, m0 ^ m1 ^ m2, range(0, 2))
        for a in range(3):
            descs[a][7].wait_recv()
        for a in range(3):
            m0, m1, m2 = ORDERS[a]
            compute(a, m0 ^ m1 ^ m2, range(2, e_loc))

        for a in range(3):
            c0, c1 = COLS[a]
            out_ref[:, c0:c1] = accs[a]

        for a in range(3):
            for fi in range(8):
                descs[a][fi].wait_send()

        @functools.partial(pl.run_scoped,
                           second_barrier=pltpu.SemaphoreType.REGULAR)
        def _(second_barrier):
            for mask in MASKS:
                pl.semaphore_signal(second_barrier, inc=1,
                                    device_id=(my ^ mask,),
                                    device_id_type=pl.DeviceIdType.MESH)
            pl.semaphore_wait(second_barrier, 3)

    return pl.pallas_call(
        body,
        out_shape=jax.ShapeDtypeStruct((n_tok, d_hidden), jnp.float32),
        in_specs=[pl.BlockSpec(memory_space=pltpu.VMEM)] * 4,
        out_specs=pl.BlockSpec(memory_space=pltpu.VMEM),
        scratch_shapes=[
            pltpu.VMEM((N_DEV, e_loc, d_model, 384), jnp.bfloat16),
            pltpu.VMEM((N_DEV, e_loc, d_model, 384), jnp.bfloat16),
            pltpu.VMEM((N_DEV, e_loc, d_model, 256), jnp.bfloat16),
            pltpu.SemaphoreType.DMA((3, 8)),
            pltpu.SemaphoreType.DMA((3, 8)),
        ],
        compiler_params=pltpu.CompilerParams(
            collective_id=0,
            vmem_limit_bytes=64 * 1024 * 1024,
        ),
    )(x_bf16, router_bf16, route_idx, w_bf16)


# device time: 139131 ns/iter; 2.5667x vs baseline; 1.0706x over previous
import functools

import jax
import jax.numpy as jnp
from jax import lax
from jax.experimental import pallas as pl
from jax.experimental.pallas import tpu as pltpu

N_DEV = 8
N_EXP = 32

MASKS = (1, 3, 4)
ORDERS = ((1, 3, 4), (3, 4, 1), (4, 1, 3))
COLS = ((0, 384), (384, 768), (768, 1024))


def kernel(x, router_W, route_idx, expert_W):
    n_tok, d_model = x.shape
    e_loc, _, d_hidden = expert_W.shape

    def body(x_ref, router_ref, idx_ref, w_ref, out_ref,
             g0_ref, g1_ref, g2_ref, send_sems, recv_sems):
        my = lax.axis_index("i")
        gs = (g0_ref, g1_ref, g2_ref)

        barrier = pltpu.get_barrier_semaphore()
        for mask in MASKS:
            pl.semaphore_signal(barrier, inc=1, device_id=(my ^ mask,),
                                device_id_type=pl.DeviceIdType.MESH)
        pl.semaphore_wait(barrier, 3)

        x_bf = x_ref[:, :].astype(jnp.bfloat16)

        scores = jnp.dot(x_bf, router_ref[:, :].astype(jnp.bfloat16),
                         preferred_element_type=jnp.float32)
        e0 = idx_ref[:, 0:1]
        e1 = idx_ref[:, 1:2]
        col = lax.broadcasted_iota(jnp.int32, (n_tok, N_EXP), 1)
        s0 = jnp.sum(jnp.where(col == e0, scores, 0.0), axis=1, keepdims=True)
        s1 = jnp.sum(jnp.where(col == e1, scores, 0.0), axis=1, keepdims=True)
        m = jnp.maximum(s0, s1)
        p0 = jnp.exp(s0 - m)
        p1 = jnp.exp(s1 - m)
        w0 = p0 / (p0 + p1)
        w1 = p1 / (p0 + p1)

        out_ref[...] = jnp.zeros((n_tok, d_hidden), jnp.float32)

        def compute(a, g, experts=range(e_loc)):
            c0, c1 = COLS[a]
            src_dev = my ^ g
            for j in experts:
                eg = src_dev * e_loc + j
                w_e = (jnp.where(e0 == eg, w0, 0.0)
                       + jnp.where(e1 == eg, w1, 0.0))
                y = jnp.dot(x_bf, gs[a][g, j],
                            preferred_element_type=jnp.float32)
                out_ref[:, c0:c1] = out_ref[:, c0:c1] + w_e * y

        descs = [[None] * 8 for _ in range(3)]

        def issue(a, step, pairs):
            mask = ORDERS[a][step]
            partner = my ^ mask
            for g, fi in pairs:
                src = gs[a].at[g]
                dst = gs[a].at[g ^ mask]
                if fi >= 6:
                    j0, j1 = (0, 2) if fi == 6 else (2, e_loc)
                    src = gs[a].at[g, j0:j1]
                    dst = gs[a].at[g ^ mask, j0:j1]
                rdma = pltpu.make_async_remote_copy(
                    src_ref=src,
                    dst_ref=dst,
                    send_sem=send_sems.at[a, fi],
                    recv_sem=recv_sems.at[a, fi],
                    device_id=(partner,),
                    device_id_type=pl.DeviceIdType.MESH,
                )
                rdma.start()
                descs[a][fi] = rdma

        for a in range(3):
            c0, c1 = COLS[a]
            gs[a][0] = w_ref[:, :, c0:c1].astype(jnp.bfloat16)
            issue(a, 0, [(0, 0)])
        for a in range(3):
            issue(a, 1, [(0, 1)])
            issue(a, 2, [(0, 3)])
        for a in range(3):
            compute(a, 0)
        for a in range(3):
            descs[a][0].wait_recv()

        for a in range(3):
            m0 = ORDERS[a][0]
            issue(a, 1, [(m0, 2)])
            issue(a, 2, [(m0, 4)])
        for a in range(3):
            compute(a, ORDERS[a][0])
        for a in range(3):
            descs[a][1].wait_recv()
            descs[a][2].wait_recv()

        for a in range(3):
            m0, m1, _ = ORDERS[a]
            issue(a, 2, [(m1, 5), (m0 ^ m1, 6), (m0 ^ m1, 7)])
        for a in range(3):
            m0, m1, _ = ORDERS[a]
            compute(a, m1)
            compute(a, m0 ^ m1)
        for a in range(3):
            descs[a][3].wait_recv()
        for a in range(3):
            compute(a, ORDERS[a][2])
        for a in range(3):
            descs[a][4].wait_recv()
        for a in range(3):
            m0, _, m2 = ORDERS[a]
            compute(a, m0 ^ m2)
        for a in range(3):
            descs[a][5].wait_recv()
        for a in range(3):
            _, m1, m2 = ORDERS[a]
            compute(a, m1 ^ m2)
        for a in range(3):
            descs[a][6].wait_recv()
        for a in range(3):
            m0, m1, m2 = ORDERS[a]
            compute(a, m0 ^ m1 ^ m2, range(0, 2))
        for a in range(3):
            descs[a][7].wait_recv()
        for a in range(3):
            m0, m1, m2 = ORDERS[a]
            compute(a, m0 ^ m1 ^ m2, range(2, e_loc))

        for a in range(3):
            for fi in range(8):
                descs[a][fi].wait_send()

        @functools.partial(pl.run_scoped,
                           second_barrier=pltpu.SemaphoreType.REGULAR)
        def _(second_barrier):
            for mask in MASKS:
                pl.semaphore_signal(second_barrier, inc=1,
                                    device_id=(my ^ mask,),
                                    device_id_type=pl.DeviceIdType.MESH)
            pl.semaphore_wait(second_barrier, 3)

    return pl.pallas_call(
        body,
        out_shape=jax.ShapeDtypeStruct((n_tok, d_hidden), jnp.float32),
        in_specs=[pl.BlockSpec(memory_space=pltpu.VMEM)] * 4,
        out_specs=pl.BlockSpec(memory_space=pltpu.VMEM),
        scratch_shapes=[
            pltpu.VMEM((N_DEV, e_loc, d_model, 384), jnp.bfloat16),
            pltpu.VMEM((N_DEV, e_loc, d_model, 384), jnp.bfloat16),
            pltpu.VMEM((N_DEV, e_loc, d_model, 256), jnp.bfloat16),
            pltpu.SemaphoreType.DMA((3, 8)),
            pltpu.SemaphoreType.DMA((3, 8)),
        ],
        compiler_params=pltpu.CompilerParams(
            collective_id=0,
            vmem_limit_bytes=64 * 1024 * 1024,
        ),
    )(x, router_W, route_idx, expert_W)


# device time: 135078 ns/iter; 2.6437x vs baseline; 1.0300x over previous
import functools

import jax
import jax.numpy as jnp
from jax import lax
from jax.experimental import pallas as pl
from jax.experimental.pallas import tpu as pltpu

N_DEV = 8
N_EXP = 32

MASKS = (1, 3, 4)
ORDERS = ((1, 3, 4), (3, 4, 1), (4, 1, 3))
PACK = (
    ((0, 0, 1024), (1, 0, 384)),
    ((1, 384, 1024), (2, 0, 768)),
    ((2, 768, 1024), (3, 0, 1024)),
)
NCOLS = tuple(sum(ce - cs for _, cs, ce in p) for p in PACK)


def kernel(x, router_W, route_idx, expert_W):
    n_tok, d_model = x.shape
    e_loc, _, d_hidden = expert_W.shape

    def body(x_ref, router_ref, idx_ref, w_ref, out_ref,
             g0_ref, g1_ref, g2_ref, send_sems, recv_sems):
        my = lax.axis_index("i")
        gs = (g0_ref, g1_ref, g2_ref)

        barrier = pltpu.get_barrier_semaphore()
        for mask in MASKS:
            pl.semaphore_signal(barrier, inc=1, device_id=(my ^ mask,),
                                device_id_type=pl.DeviceIdType.MESH)
        pl.semaphore_wait(barrier, 3)

        descs = [[None] * 7 for _ in range(3)]

        def issue(a, step, pairs):
            mask = ORDERS[a][step]
            partner = my ^ mask
            for g, fi in pairs:
                rdma = pltpu.make_async_remote_copy(
                    src_ref=gs[a].at[g],
                    dst_ref=gs[a].at[g ^ mask],
                    send_sem=send_sems.at[a, fi],
                    recv_sem=recv_sems.at[a, fi],
                    device_id=(partner,),
                    device_id_type=pl.DeviceIdType.MESH,
                )
                rdma.start()
                descs[a][fi] = rdma

        for a in range(3):
            off = 0
            for j, cs, ce in PACK[a]:
                w = ce - cs
                gs[a][0, :, off:off + w] = (
                    w_ref[j, :, cs:ce].astype(jnp.bfloat16))
                off += w
            issue(a, 0, [(0, 0)])
        for a in range(3):
            issue(a, 1, [(0, 1)])
            issue(a, 2, [(0, 3)])

        x_bf = x_ref[:, :].astype(jnp.bfloat16)

        scores = jnp.dot(x_bf, router_ref[:, :].astype(jnp.bfloat16),
                         preferred_element_type=jnp.float32)
        e0 = idx_ref[:, 0:1]
        e1 = idx_ref[:, 1:2]
        col = lax.broadcasted_iota(jnp.int32, (n_tok, N_EXP), 1)
        s0 = jnp.sum(jnp.where(col == e0, scores, 0.0), axis=1, keepdims=True)
        s1 = jnp.sum(jnp.where(col == e1, scores, 0.0), axis=1, keepdims=True)
        m = jnp.maximum(s0, s1)
        p0 = jnp.exp(s0 - m)
        p1 = jnp.exp(s1 - m)
        w0 = p0 / (p0 + p1)
        w1 = p1 / (p0 + p1)

        out_ref[...] = jnp.zeros((n_tok, d_hidden), jnp.float32)

        def compute(a, g):
            src_dev = my ^ g
            off = 0
            for j, cs, ce in PACK[a]:
                w = ce - cs
                eg = src_dev * e_loc + j
                w_e = (jnp.where(e0 == eg, w0, 0.0)
                       + jnp.where(e1 == eg, w1, 0.0))
                y = jnp.dot(x_bf, gs[a][g, :, off:off + w],
                            preferred_element_type=jnp.float32)
                out_ref[:, cs:ce] = out_ref[:, cs:ce] + w_e * y
                off += w

        for a in range(3):
            compute(a, 0)
        for a in range(3):
            descs[a][0].wait_recv()

        for a in range(3):
            m0 = ORDERS[a][0]
            issue(a, 1, [(m0, 2)])
            issue(a, 2, [(m0, 4)])
        for a in range(3):
            compute(a, ORDERS[a][0])
        for a in range(3):
            descs[a][1].wait_recv()
            descs[a][2].wait_recv()

        for a in range(3):
            m0, m1, _ = ORDERS[a]
            issue(a, 2, [(m1, 5), (m0 ^ m1, 6)])
        for a in range(3):
            m0, m1, _ = ORDERS[a]
            compute(a, m1)
            compute(a, m0 ^ m1)
        for a in range(3):
            descs[a][3].wait_recv()
        for a in range(3):
            compute(a, ORDERS[a][2])
        for a in range(3):
            descs[a][4].wait_recv()
        for a in range(3):
            m0, _, m2 = ORDERS[a]
            compute(a, m0 ^ m2)
        for a in range(3):
            descs[a][5].wait_recv()
        for a in range(3):
            _, m1, m2 = ORDERS[a]
            compute(a, m1 ^ m2)
        for a in range(3):
            descs[a][6].wait_recv()
        for a in range(3):
            m0, m1, m2 = ORDERS[a]
            compute(a, m0 ^ m1 ^ m2)

        for a in range(3):
            for fi in range(7):
                descs[a][fi].wait_send()

        @functools.partial(pl.run_scoped,
                           second_barrier=pltpu.SemaphoreType.REGULAR)
        def _(second_barrier):
            for mask in MASKS:
                pl.semaphore_signal(second_barrier, inc=1,
                                    device_id=(my ^ mask,),
                                    device_id_type=pl.DeviceIdType.MESH)
            pl.semaphore_wait(second_barrier, 3)

    return pl.pallas_call(
        body,
        out_shape=jax.ShapeDtypeStruct((n_tok, d_hidden), jnp.float32),
        in_specs=[pl.BlockSpec(memory_space=pltpu.VMEM)] * 4,
        out_specs=pl.BlockSpec(memory_space=pltpu.VMEM),
        scratch_shapes=[
            pltpu.VMEM((N_DEV, d_model, NCOLS[0]), jnp.bfloat16),
            pltpu.VMEM((N_DEV, d_model, NCOLS[1]), jnp.bfloat16),
            pltpu.VMEM((N_DEV, d_model, NCOLS[2]), jnp.bfloat16),
            pltpu.SemaphoreType.DMA((3, 7)),
            pltpu.SemaphoreType.DMA((3, 7)),
        ],
        compiler_params=pltpu.CompilerParams(
            collective_id=0,
            vmem_limit_bytes=64 * 1024 * 1024,
        ),
    )(x, router_W, route_idx, expert_W)
